# Initial kernel scaffold; baseline (speedup 1.0000x reference)
import jax
import jax.numpy as jnp
from jax.experimental import pallas as pl


def kernel(s_hidden, t_hidden, s_input_grad, t_input_grad):
    raise NotImplementedError("write your pallas kernel here")


# speedup vs baseline: 3.8569x; 3.8569x over previous
import functools

import jax
import jax.numpy as jnp
from jax.experimental import pallas as pl
from jax.experimental.pallas import tpu as pltpu

TOP_K = 64
EPS = 1e-12
ROWS = 512


def _body(sh_ref, th_ref, sg_ref, tg_ref, out_ref, acc_ref):
    b = pl.program_id(0)
    j = pl.program_id(1)
    nb = pl.num_programs(0)
    nj = pl.num_programs(1)

    @pl.when((b == 0) & (j == 0))
    def _init():
        acc_ref[...] = jnp.zeros_like(acc_ref)

    t = th_ref[0] * tg_ref[0]
    v = t * t
    bits = jax.lax.bitcast_convert_type(v, jnp.int32)

    lo = jnp.zeros((ROWS, 1), jnp.int32)
    hi = jnp.full((ROWS, 1), 0x7F7FFFFF, jnp.int32)

    def search(_, lh):
        lo, hi = lh
        mid = lo + (hi - lo + 1) // 2
        cnt = jnp.sum((bits >= mid).astype(jnp.int32), axis=1, keepdims=True)
        ge = cnt >= TOP_K
        return jnp.where(ge, mid, lo), jnp.where(ge, hi, mid - 1)

    lo, hi = jax.lax.fori_loop(0, 31, search, (lo, hi))
    tau = jax.lax.bitcast_convert_type(lo, jnp.float32)

    gt = bits > lo
    cnt_gt = jnp.sum(jnp.where(gt, 1.0, 0.0), axis=1, keepdims=True)
    sum_gt = jnp.sum(jnp.where(gt, v, 0.0), axis=1, keepdims=True)
    S = sum_gt + (TOP_K - cnt_gt) * tau
    t_row = jnp.sqrt(S)

    s = sh_ref[0] * sg_ref[0]
    s_sq = jnp.sum(s * s, axis=1, keepdims=True)
    s_row = jnp.sqrt(s_sq)

    pA = jnp.sum(S).reshape(1, 1)
    pB = jnp.sum(s_sq).reshape(1, 1)
    pC = jnp.sum(t_row * s_row).reshape(1, 1)
    pD = jnp.sum(jnp.where(S > 0, 1.0, 0.0)).reshape(1, 1)

    for idx, val in enumerate((pA, pB, pC, pD)):
        acc_ref[pl.ds(b, 1), pl.ds(idx, 1)] += val

    @pl.when((b == nb - 1) & (j == nj - 1))
    def _finish():
        total = jnp.zeros((1, 1), jnp.float32)
        denom = jnp.zeros((1, 1), jnp.float32)
        for bb in range(2):
            A = acc_ref[bb : bb + 1, 0:1]
            B = acc_ref[bb : bb + 1, 1:2]
            C = acc_ref[bb : bb + 1, 2:3]
            D = acc_ref[bb : bb + 1, 3:4]
            nt = jnp.maximum(jnp.sqrt(A), EPS)
            ns = jnp.maximum(jnp.sqrt(B), EPS)
            total += A / (nt * nt) + B / (ns * ns) - 2.0 * C / (nt * ns)
            denom += D
        out_ref[...] = total / denom


@jax.jit
def kernel(s_hidden, t_hidden, s_input_grad, t_input_grad):
    batch, seq, dim = t_hidden.shape
    grid = (batch, seq // ROWS)
    spec = pl.BlockSpec((1, ROWS, dim), lambda b, j: (b, j, 0))
    out = pl.pallas_call(
        _body,
        grid=grid,
        in_specs=[spec, spec, spec, spec],
        out_specs=pl.BlockSpec((1, 1), lambda b, j: (0, 0)),
        out_shape=jax.ShapeDtypeStruct((1, 1), jnp.float32),
        scratch_shapes=[pltpu.VMEM((2, 4), jnp.float32)],
    )(s_hidden, t_hidden, s_input_grad, t_input_grad)
    return out[0, 0]


# speedup vs baseline: 6.0156x; 1.5597x over previous
import functools

import jax
import jax.numpy as jnp
from jax.experimental import pallas as pl
from jax.experimental.pallas import tpu as pltpu

TOP_K = 64
EPS = 1e-12
ROWS = 512


def _body(sh_ref, th_ref, sg_ref, tg_ref, out_ref, acc_ref):
    b = pl.program_id(0)
    j = pl.program_id(1)
    nb = pl.num_programs(0)
    nj = pl.num_programs(1)

    @pl.when((b == 0) & (j == 0))
    def _init():
        acc_ref[...] = jnp.zeros_like(acc_ref)

    t = th_ref[0] * tg_ref[0]
    v = t * t
    key = jax.lax.shift_right_logical(
        jax.lax.bitcast_convert_type(v, jnp.int32), 13
    )

    lo = jnp.zeros((ROWS, 1), jnp.int32)
    hi = jnp.full((ROWS, 1), 0x7F7FFFFF >> 13, jnp.int32)

    def search(_, lh):
        lo, hi = lh
        mid = lo + (hi - lo + 1) // 2
        cnt = jnp.sum((key >= mid).astype(jnp.int32), axis=1, keepdims=True)
        ge = cnt >= TOP_K
        return jnp.where(ge, mid, lo), jnp.where(ge, hi, mid - 1)

    lo, hi = jax.lax.fori_loop(0, 18, search, (lo, hi))

    gt = key > lo
    eq = key == lo
    cnt_gt = jnp.sum(jnp.where(gt, 1.0, 0.0), axis=1, keepdims=True)
    sum_gt = jnp.sum(jnp.where(gt, v, 0.0), axis=1, keepdims=True)
    cnt_eq = jnp.sum(jnp.where(eq, 1.0, 0.0), axis=1, keepdims=True)
    sum_eq = jnp.sum(jnp.where(eq, v, 0.0), axis=1, keepdims=True)
    S = sum_gt + (TOP_K - cnt_gt) * (sum_eq / cnt_eq)
    t_row = jnp.sqrt(S)

    s = sh_ref[0] * sg_ref[0]
    s_sq = jnp.sum(s * s, axis=1, keepdims=True)
    s_row = jnp.sqrt(s_sq)

    pA = jnp.sum(S).reshape(1, 1)
    pB = jnp.sum(s_sq).reshape(1, 1)
    pC = jnp.sum(t_row * s_row).reshape(1, 1)
    pD = jnp.sum(jnp.where(S > 0, 1.0, 0.0)).reshape(1, 1)

    for idx, val in enumerate((pA, pB, pC, pD)):
        acc_ref[pl.ds(b, 1), pl.ds(idx, 1)] += val

    @pl.when((b == nb - 1) & (j == nj - 1))
    def _finish():
        total = jnp.zeros((1, 1), jnp.float32)
        denom = jnp.zeros((1, 1), jnp.float32)
        for bb in range(2):
            A = acc_ref[bb : bb + 1, 0:1]
            B = acc_ref[bb : bb + 1, 1:2]
            C = acc_ref[bb : bb + 1, 2:3]
            D = acc_ref[bb : bb + 1, 3:4]
            nt = jnp.maximum(jnp.sqrt(A), EPS)
            ns = jnp.maximum(jnp.sqrt(B), EPS)
            total += A / (nt * nt) + B / (ns * ns) - 2.0 * C / (nt * ns)
            denom += D
        out_ref[...] = total / denom


@jax.jit
def kernel(s_hidden, t_hidden, s_input_grad, t_input_grad):
    batch, seq, dim = t_hidden.shape
    grid = (batch, seq // ROWS)
    spec = pl.BlockSpec((1, ROWS, dim), lambda b, j: (b, j, 0))
    out = pl.pallas_call(
        _body,
        grid=grid,
        in_specs=[spec, spec, spec, spec],
        out_specs=pl.BlockSpec((1, 1), lambda b, j: (0, 0)),
        out_shape=jax.ShapeDtypeStruct((1, 1), jnp.float32),
        scratch_shapes=[pltpu.VMEM((2, 4), jnp.float32)],
    )(s_hidden, t_hidden, s_input_grad, t_input_grad)
    return out[0, 0]
